# baseline (speedup 1.0000x reference)
import functools

import jax
import jax.numpy as jnp
from jax import lax
from jax.experimental import pallas as pl
from jax.experimental.pallas import tpu as pltpu
from jax.experimental.pallas import tpu_sc as plsc

_N = 2048
_P = 131072
_B = 8
_SEG = _P // _B
_NC = 2
_NS = 16
_L = 16
_NW = _NC * _NS
_PPW = _P // _NW
_VECS = _PPW // _L
_SEG_PER_CORE = _B // _NC
_W_PER_SEG = _SEG // _PPW


def _tc_table_body(r_ref, xyz_ref, sW1, sb1, sW2, sb2, sW3, sb3,
                   eW1, eb1, eW2, eb2, eW3, eb3, out_ref):
    rT = r_ref[...].T
    cdims = (((0,), (0,)), ((), ()))

    h = jnp.tanh(lax.dot_general(sW1[...], rT, cdims,
                                 preferred_element_type=jnp.float32) + sb1[...])
    h = jnp.tanh(lax.dot_general(sW2[...], h, cdims,
                                 preferred_element_type=jnp.float32) + sb2[...])
    s_out = lax.dot_general(sW3[...], h, cdims,
                            preferred_element_type=jnp.float32) + sb3[...]
    ssig = jnp.sqrt(4.0 + 10.0 * s_out * s_out)

    g = jnp.tanh(lax.dot_general(eW1[...], rT, cdims,
                                 preferred_element_type=jnp.float32) + eb1[...])
    g = jnp.tanh(lax.dot_general(eW2[...], g, cdims,
                                 preferred_element_type=jnp.float32) + eb2[...])
    e_out = lax.dot_general(eW3[...], g, cdims,
                            preferred_element_type=jnp.float32) + eb3[...]
    seps = jnp.sqrt(0.1) * jnp.abs(e_out)

    out_ref[...] = jnp.concatenate([xyz_ref[...].T, ssig, seps], axis=0)


def _sc_pairs_body(tbl_hbm, pr_hbm, out_hbm,
                   tbl_v, pr_v, acc_v, shared_s, loc_v, out_v,
                   sem0, sem1):
    c = lax.axis_index("c")
    s = lax.axis_index("s")
    base = (c * _NS + s) * _PPW

    cp0 = pltpu.async_copy(tbl_hbm, tbl_v, sem0)
    cp1 = pltpu.async_copy(pr_hbm.at[pl.ds(base * 2, _PPW * 2)], pr_v, sem1)
    cp0.wait()
    cp1.wait()

    half = jnp.full((_L,), 0.5, jnp.float32)
    three_half = jnp.full((_L,), 1.5, jnp.float32)
    magic = jnp.full((_L,), 0x5F3759DF, jnp.int32)
    iota2 = lax.iota(jnp.int32, _L) * 2

    def body(i, acc):
        rows2 = i * (2 * _L) + iota2
        i0 = plsc.load_gather(pr_v, [rows2])
        i1 = plsc.load_gather(pr_v, [rows2 + 1])
        x0 = plsc.load_gather(tbl_v, [i0])
        y0 = plsc.load_gather(tbl_v, [i0 + _N])
        z0 = plsc.load_gather(tbl_v, [i0 + 2 * _N])
        sg0 = plsc.load_gather(tbl_v, [i0 + 3 * _N])
        ep0 = plsc.load_gather(tbl_v, [i0 + 4 * _N])
        x1 = plsc.load_gather(tbl_v, [i1])
        y1 = plsc.load_gather(tbl_v, [i1 + _N])
        z1 = plsc.load_gather(tbl_v, [i1 + 2 * _N])
        sg1 = plsc.load_gather(tbl_v, [i1 + 3 * _N])
        ep1 = plsc.load_gather(tbl_v, [i1 + 4 * _N])

        dx = x1 - x0
        dy = y1 - y0
        dz = z1 - z0
        d2 = dx * dx + dy * dy + dz * dz
        ib = magic - lax.shift_right_arithmetic(plsc.bitcast(d2, jnp.int32), 1)
        y = plsc.bitcast(ib, jnp.float32)
        hd2 = half * d2
        y = y * (three_half - hd2 * y * y)
        y = y * (three_half - hd2 * y * y)

        t = (sg0 * sg1) * y
        t2 = t * t
        t6 = t2 * t2 * t2
        em4 = 4.0 * (ep0 * ep1)
        return acc + em4 * (t6 * t6 - t6)

    acc = lax.fori_loop(0, _VECS, body, jnp.zeros((_L,), jnp.float32))
    acc_v[...] = acc
    pltpu.sync_copy(acc_v, shared_s.at[pl.ds(s * _L, _L)])
    plsc.subcore_barrier()

    @pl.when(s == 0)
    def _():
        pltpu.sync_copy(shared_s, loc_v)
        iot = lax.iota(jnp.int32, _L)
        outv = jnp.zeros((_L,), jnp.float32)
        for seg in range(_SEG_PER_CORE):
            a = loc_v[pl.ds((seg * _W_PER_SEG + 0) * _L, _L)]
            for j in range(1, _W_PER_SEG):
                a = a + loc_v[pl.ds((seg * _W_PER_SEG + j) * _L, _L)]
            tot = jnp.sum(a)
            outv = jnp.where(iot == seg, tot, outv)
        out_v[...] = outv
        pltpu.sync_copy(out_v, out_hbm.at[c])


@jax.jit
def _impl(r, pairs, xyz, s_W1, s_b1, s_W2, s_b2, s_W3, s_b3,
          e_W1, e_b1, e_W2, e_b2, e_W3, e_b3):
    tc_table = pl.pallas_call(
        _tc_table_body,
        out_shape=jax.ShapeDtypeStruct((5, _N), jnp.float32),
    )
    tbl = tc_table(
        r, xyz,
        s_W1, s_b1.reshape(-1, 1), s_W2, s_b2.reshape(-1, 1),
        s_W3, s_b3.reshape(-1, 1),
        e_W1, e_b1.reshape(-1, 1), e_W2, e_b2.reshape(-1, 1),
        e_W3, e_b3.reshape(-1, 1),
    ).reshape(-1)

    sc_pairs = functools.partial(
        pl.kernel,
        out_type=jax.ShapeDtypeStruct((_NC, _L), jnp.float32),
        mesh=plsc.VectorSubcoreMesh(core_axis_name="c", subcore_axis_name="s"),
        compiler_params=pltpu.CompilerParams(needs_layout_passes=False),
        scratch_types=[
            pltpu.VMEM((_N * 5,), jnp.float32),
            pltpu.VMEM((_PPW * 2,), jnp.int32),
            pltpu.VMEM((_L,), jnp.float32),
            pltpu.VMEM_SHARED((_NS * _L,), jnp.float32),
            pltpu.VMEM((_NS * _L,), jnp.float32),
            pltpu.VMEM((_L,), jnp.float32),
            pltpu.SemaphoreType.DMA,
            pltpu.SemaphoreType.DMA,
        ],
    )(_sc_pairs_body)

    out2 = sc_pairs(tbl, pairs.astype(jnp.int32).reshape(-1))
    return jnp.concatenate(
        [out2[0, :_SEG_PER_CORE], out2[1, :_SEG_PER_CORE]]).reshape(-1, 1)


def kernel(r, pairs, num_pairs, xyz, s_W1, s_b1, s_W2, s_b2, s_W3, s_b3,
           e_W1, e_b1, e_W2, e_b2, e_W3, e_b3):
    del num_pairs
    return _impl(r, pairs, xyz, s_W1, s_b1, s_W2, s_b2, s_W3, s_b3,
                 e_W1, e_b1, e_W2, e_b2, e_W3, e_b3)


# speedup vs baseline: 2.7798x; 2.7798x over previous
import functools

import jax
import jax.numpy as jnp
from jax import lax
from jax.experimental import pallas as pl
from jax.experimental.pallas import tpu as pltpu
from jax.experimental.pallas import tpu_sc as plsc

_N = 2048
_P = 131072
_B = 8
_SEG = _P // _B
_NC = 2
_NS = 16
_L = 16
_NW = _NC * _NS
_PPW = _P // _NW
_VECS = _PPW // _L
_SEG_PER_CORE = _B // _NC
_W_PER_SEG = _SEG // _PPW


def _tc_table_body(r_ref, xyz_ref, sW1, sb1, sW2, sb2, sW3, sb3,
                   eW1, eb1, eW2, eb2, eW3, eb3, out_ref):
    rT = r_ref[...].T
    cdims = (((0,), (0,)), ((), ()))

    h = jnp.tanh(lax.dot_general(sW1[...], rT, cdims,
                                 preferred_element_type=jnp.float32) + sb1[...])
    h = jnp.tanh(lax.dot_general(sW2[...], h, cdims,
                                 preferred_element_type=jnp.float32) + sb2[...])
    s_out = lax.dot_general(sW3[...], h, cdims,
                            preferred_element_type=jnp.float32) + sb3[...]
    ssig = jnp.sqrt(4.0 + 10.0 * s_out * s_out)

    g = jnp.tanh(lax.dot_general(eW1[...], rT, cdims,
                                 preferred_element_type=jnp.float32) + eb1[...])
    g = jnp.tanh(lax.dot_general(eW2[...], g, cdims,
                                 preferred_element_type=jnp.float32) + eb2[...])
    e_out = lax.dot_general(eW3[...], g, cdims,
                            preferred_element_type=jnp.float32) + eb3[...]
    seps = jnp.sqrt(0.1) * jnp.abs(e_out)

    out_ref[...] = jnp.concatenate([xyz_ref[...].T, ssig, seps], axis=0)


def _sc_pairs_body(tbl_hbm, p0_hbm, p1_hbm, out_hbm,
                   tbl_v, p0_v, p1_v, acc_v, shared_s, loc_v, out_v,
                   sem0, sem1, sem2):
    c = lax.axis_index("c")
    s = lax.axis_index("s")
    base = (c * _NS + s) * _PPW

    cp0 = pltpu.async_copy(tbl_hbm, tbl_v, sem0)
    cp1 = pltpu.async_copy(p0_hbm.at[pl.ds(base, _PPW)], p0_v, sem1)
    cp2 = pltpu.async_copy(p1_hbm.at[pl.ds(base, _PPW)], p1_v, sem2)
    cp0.wait()
    cp1.wait()
    cp2.wait()

    half = jnp.full((_L,), 0.5, jnp.float32)
    three_half = jnp.full((_L,), 1.5, jnp.float32)
    magic = jnp.full((_L,), 0x5F3759DF, jnp.int32)
    def body(i, acc):
        i0 = p0_v[pl.ds(i * _L, _L)]
        i1 = p1_v[pl.ds(i * _L, _L)]
        x0 = plsc.load_gather(tbl_v, [i0])
        y0 = plsc.load_gather(tbl_v, [i0 + _N])
        z0 = plsc.load_gather(tbl_v, [i0 + 2 * _N])
        sg0 = plsc.load_gather(tbl_v, [i0 + 3 * _N])
        ep0 = plsc.load_gather(tbl_v, [i0 + 4 * _N])
        x1 = plsc.load_gather(tbl_v, [i1])
        y1 = plsc.load_gather(tbl_v, [i1 + _N])
        z1 = plsc.load_gather(tbl_v, [i1 + 2 * _N])
        sg1 = plsc.load_gather(tbl_v, [i1 + 3 * _N])
        ep1 = plsc.load_gather(tbl_v, [i1 + 4 * _N])

        dx = x1 - x0
        dy = y1 - y0
        dz = z1 - z0
        d2 = dx * dx + dy * dy + dz * dz
        ib = magic - lax.shift_right_arithmetic(plsc.bitcast(d2, jnp.int32), 1)
        y = plsc.bitcast(ib, jnp.float32)
        hd2 = half * d2
        y = y * (three_half - hd2 * y * y)
        y = y * (three_half - hd2 * y * y)

        t = (sg0 * sg1) * y
        t2 = t * t
        t6 = t2 * t2 * t2
        em4 = 4.0 * (ep0 * ep1)
        return acc + em4 * (t6 * t6 - t6)

    acc = lax.fori_loop(0, _VECS, body, jnp.zeros((_L,), jnp.float32))
    acc_v[...] = acc
    pltpu.sync_copy(acc_v, shared_s.at[pl.ds(s * _L, _L)])
    plsc.subcore_barrier()

    @pl.when(s == 0)
    def _():
        pltpu.sync_copy(shared_s, loc_v)
        iot = lax.iota(jnp.int32, _L)
        outv = jnp.zeros((_L,), jnp.float32)
        for seg in range(_SEG_PER_CORE):
            a = loc_v[pl.ds((seg * _W_PER_SEG + 0) * _L, _L)]
            for j in range(1, _W_PER_SEG):
                a = a + loc_v[pl.ds((seg * _W_PER_SEG + j) * _L, _L)]
            tot = jnp.sum(a)
            outv = jnp.where(iot == seg, tot, outv)
        out_v[...] = outv
        pltpu.sync_copy(out_v, out_hbm.at[c])


@jax.jit
def _impl(r, pairs, xyz, s_W1, s_b1, s_W2, s_b2, s_W3, s_b3,
          e_W1, e_b1, e_W2, e_b2, e_W3, e_b3):
    tc_table = pl.pallas_call(
        _tc_table_body,
        out_shape=jax.ShapeDtypeStruct((5, _N), jnp.float32),
    )
    tbl = tc_table(
        r, xyz,
        s_W1, s_b1.reshape(-1, 1), s_W2, s_b2.reshape(-1, 1),
        s_W3, s_b3.reshape(-1, 1),
        e_W1, e_b1.reshape(-1, 1), e_W2, e_b2.reshape(-1, 1),
        e_W3, e_b3.reshape(-1, 1),
    ).reshape(-1)

    sc_pairs = functools.partial(
        pl.kernel,
        out_type=jax.ShapeDtypeStruct((_NC, _L), jnp.float32),
        mesh=plsc.VectorSubcoreMesh(core_axis_name="c", subcore_axis_name="s"),
        compiler_params=pltpu.CompilerParams(needs_layout_passes=False),
        scratch_types=[
            pltpu.VMEM((_N * 5,), jnp.float32),
            pltpu.VMEM((_PPW,), jnp.int32),
            pltpu.VMEM((_PPW,), jnp.int32),
            pltpu.VMEM((_L,), jnp.float32),
            pltpu.VMEM_SHARED((_NS * _L,), jnp.float32),
            pltpu.VMEM((_NS * _L,), jnp.float32),
            pltpu.VMEM((_L,), jnp.float32),
            pltpu.SemaphoreType.DMA,
            pltpu.SemaphoreType.DMA,
            pltpu.SemaphoreType.DMA,
        ],
    )(_sc_pairs_body)

    p0 = pairs[:, 0].astype(jnp.int32)
    p1 = pairs[:, 1].astype(jnp.int32)
    out2 = sc_pairs(tbl, p0, p1)
    return jnp.concatenate(
        [out2[0, :_SEG_PER_CORE], out2[1, :_SEG_PER_CORE]]).reshape(-1, 1)


def kernel(r, pairs, num_pairs, xyz, s_W1, s_b1, s_W2, s_b2, s_W3, s_b3,
           e_W1, e_b1, e_W2, e_b2, e_W3, e_b3):
    del num_pairs
    return _impl(r, pairs, xyz, s_W1, s_b1, s_W2, s_b2, s_W3, s_b3,
                 e_W1, e_b1, e_W2, e_b2, e_W3, e_b3)


# speedup vs baseline: 3.2074x; 1.1538x over previous
import functools

import jax
import jax.numpy as jnp
from jax import lax
from jax.experimental import pallas as pl
from jax.experimental.pallas import tpu as pltpu
from jax.experimental.pallas import tpu_sc as plsc

_N = 2048
_P = 131072
_B = 8
_SEG = _P // _B
_NC = 2
_NS = 16
_L = 16
_NW = _NC * _NS
_PPW = _P // _NW
_VECS = _PPW // _L
_SEG_PER_CORE = _B // _NC
_W_PER_SEG = _SEG // _PPW


def _tc_table_body(r_ref, xyz_ref, sW1, sb1, sW2, sb2, sW3, sb3,
                   eW1, eb1, eW2, eb2, eW3, eb3, out_ref):
    r = r_ref[...]
    h = jnp.tanh(jnp.dot(r, sW1[...], preferred_element_type=jnp.float32) + sb1[...])
    h = jnp.tanh(jnp.dot(h, sW2[...], preferred_element_type=jnp.float32) + sb2[...])
    s_out = jnp.dot(h, sW3[...], preferred_element_type=jnp.float32) + sb3[...]
    ssig = jnp.sqrt(4.0 + 10.0 * s_out * s_out)

    g = jnp.tanh(jnp.dot(r, eW1[...], preferred_element_type=jnp.float32) + eb1[...])
    g = jnp.tanh(jnp.dot(g, eW2[...], preferred_element_type=jnp.float32) + eb2[...])
    e_out = jnp.dot(g, eW3[...], preferred_element_type=jnp.float32) + eb3[...]
    seps = jnp.sqrt(0.1) * jnp.abs(e_out)

    out_ref[...] = jnp.concatenate(
        [xyz_ref[...].T, ssig.T, seps.T], axis=0)


def _sc_pairs_body(tbl_hbm, p0_hbm, p1_hbm, out_hbm,
                   tbl_v, p0_v, p1_v, acc_v, shared_s, loc_v, out_v,
                   sem0, sem1, sem2):
    c = lax.axis_index("c")
    s = lax.axis_index("s")
    base = (c * _NS + s) * _PPW

    cp0 = pltpu.async_copy(tbl_hbm, tbl_v, sem0)
    cp1 = pltpu.async_copy(p0_hbm.at[pl.ds(base, _PPW)], p0_v, sem1)
    cp2 = pltpu.async_copy(p1_hbm.at[pl.ds(base, _PPW)], p1_v, sem2)
    cp0.wait()
    cp1.wait()
    cp2.wait()

    half = jnp.full((_L,), 0.5, jnp.float32)
    three_half = jnp.full((_L,), 1.5, jnp.float32)
    magic = jnp.full((_L,), 0x5F3759DF, jnp.int32)
    def body(i, acc):
        i0 = p0_v[pl.ds(i * _L, _L)]
        i1 = p1_v[pl.ds(i * _L, _L)]
        x0 = plsc.load_gather(tbl_v, [i0])
        y0 = plsc.load_gather(tbl_v, [i0 + _N])
        z0 = plsc.load_gather(tbl_v, [i0 + 2 * _N])
        sg0 = plsc.load_gather(tbl_v, [i0 + 3 * _N])
        ep0 = plsc.load_gather(tbl_v, [i0 + 4 * _N])
        x1 = plsc.load_gather(tbl_v, [i1])
        y1 = plsc.load_gather(tbl_v, [i1 + _N])
        z1 = plsc.load_gather(tbl_v, [i1 + 2 * _N])
        sg1 = plsc.load_gather(tbl_v, [i1 + 3 * _N])
        ep1 = plsc.load_gather(tbl_v, [i1 + 4 * _N])

        dx = x1 - x0
        dy = y1 - y0
        dz = z1 - z0
        d2 = dx * dx + dy * dy + dz * dz
        ib = magic - lax.shift_right_arithmetic(plsc.bitcast(d2, jnp.int32), 1)
        y = plsc.bitcast(ib, jnp.float32)
        hd2 = half * d2
        y = y * (three_half - hd2 * y * y)
        y = y * (three_half - hd2 * y * y)

        t = (sg0 * sg1) * y
        t2 = t * t
        t6 = t2 * t2 * t2
        em4 = 4.0 * (ep0 * ep1)
        return acc + em4 * (t6 * t6 - t6)

    acc = lax.fori_loop(0, _VECS, body, jnp.zeros((_L,), jnp.float32))
    acc_v[...] = acc
    pltpu.sync_copy(acc_v, shared_s.at[pl.ds(s * _L, _L)])
    plsc.subcore_barrier()

    @pl.when(s == 0)
    def _():
        pltpu.sync_copy(shared_s, loc_v)
        iot = lax.iota(jnp.int32, _L)
        outv = jnp.zeros((_L,), jnp.float32)
        for seg in range(_SEG_PER_CORE):
            a = loc_v[pl.ds((seg * _W_PER_SEG + 0) * _L, _L)]
            for j in range(1, _W_PER_SEG):
                a = a + loc_v[pl.ds((seg * _W_PER_SEG + j) * _L, _L)]
            tot = jnp.sum(a)
            outv = jnp.where(iot == seg, tot, outv)
        out_v[...] = outv
        pltpu.sync_copy(out_v, out_hbm.at[c])


@jax.jit
def _impl(r, pairs, xyz, s_W1, s_b1, s_W2, s_b2, s_W3, s_b3,
          e_W1, e_b1, e_W2, e_b2, e_W3, e_b3):
    tc_table = pl.pallas_call(
        _tc_table_body,
        out_shape=jax.ShapeDtypeStruct((5, _N), jnp.float32),
    )
    tbl = tc_table(
        r, xyz,
        s_W1, s_b1.reshape(1, -1), s_W2, s_b2.reshape(1, -1),
        s_W3, s_b3.reshape(1, -1),
        e_W1, e_b1.reshape(1, -1), e_W2, e_b2.reshape(1, -1),
        e_W3, e_b3.reshape(1, -1),
    ).reshape(-1)

    sc_pairs = functools.partial(
        pl.kernel,
        out_type=jax.ShapeDtypeStruct((_NC, _L), jnp.float32),
        mesh=plsc.VectorSubcoreMesh(core_axis_name="c", subcore_axis_name="s"),
        compiler_params=pltpu.CompilerParams(needs_layout_passes=False),
        scratch_types=[
            pltpu.VMEM((_N * 5,), jnp.float32),
            pltpu.VMEM((_PPW,), jnp.int32),
            pltpu.VMEM((_PPW,), jnp.int32),
            pltpu.VMEM((_L,), jnp.float32),
            pltpu.VMEM_SHARED((_NS * _L,), jnp.float32),
            pltpu.VMEM((_NS * _L,), jnp.float32),
            pltpu.VMEM((_L,), jnp.float32),
            pltpu.SemaphoreType.DMA,
            pltpu.SemaphoreType.DMA,
            pltpu.SemaphoreType.DMA,
        ],
    )(_sc_pairs_body)

    p0 = pairs[:, 0].astype(jnp.int32)
    p1 = pairs[:, 1].astype(jnp.int32)
    out2 = sc_pairs(tbl, p0, p1)
    return jnp.concatenate(
        [out2[0, :_SEG_PER_CORE], out2[1, :_SEG_PER_CORE]]).reshape(-1, 1)


def kernel(r, pairs, num_pairs, xyz, s_W1, s_b1, s_W2, s_b2, s_W3, s_b3,
           e_W1, e_b1, e_W2, e_b2, e_W3, e_b3):
    del num_pairs
    return _impl(r, pairs, xyz, s_W1, s_b1, s_W2, s_b2, s_W3, s_b3,
                 e_W1, e_b1, e_W2, e_b2, e_W3, e_b3)
